# baseline (speedup 1.0000x reference)
import functools

import jax
import jax.numpy as jnp
from jax import lax
from jax.experimental import pallas as pl
from jax.experimental.pallas import tpu as pltpu
from jax.experimental.pallas import tpu_sc as plsc

DIM = 32
L = 16
NC = 2
NS = 16
NW = NC * NS
BATCH = 16384
BPW = BATCH // NW
CHUNK = 128
NCHUNK = BPW // CHUNK


_GATHER_DNUMS = lax.GatherDimensionNumbers(
    offset_dims=(), collapsed_slice_dims=(0,), start_index_map=(0,))


def _permute(x, idx):
    return lax.gather(x, idx[:, None], _GATHER_DNUMS, (1,),
                      indices_are_sorted=False, unique_indices=False,
                      mode=lax.GatherScatterMode.PROMISE_IN_BOUNDS)


def _vreduce_splat(v, lane):
    for sh in (8, 4, 2, 1):
        v = v + _permute(v, lane ^ sh)
    return v


def _rsqrt_nr(x):
    i = lax.bitcast_convert_type(x, jnp.int32)
    i = jnp.int32(0x5F3759DF) - lax.shift_right_logical(i, 1)
    y = lax.bitcast_convert_type(i, jnp.float32)
    for _ in range(3):
        y = y * (jnp.float32(1.5) - jnp.float32(0.5) * x * y * y)
    return y


def _clamp_scale(n):
    return jnp.minimum(jnp.float32(1.0), _rsqrt_nr(n))


def _make_sc_kernel():
    mesh = plsc.VectorSubcoreMesh(core_axis_name="c", subcore_axis_name="s")

    @functools.partial(
        pl.kernel,
        mesh=mesh,
        out_type=jax.ShapeDtypeStruct((BATCH,), jnp.float32),
        compiler_params=pltpu.CompilerParams(use_tc_tiling_on_sc=False),
        scratch_types=[
            pltpu.VMEM((NCHUNK, CHUNK), jnp.int32),
            pltpu.VMEM((NCHUNK, CHUNK), jnp.int32),
            pltpu.VMEM((NCHUNK, CHUNK), jnp.int32),
            pltpu.VMEM((BPW, DIM), jnp.float32),
            pltpu.VMEM((BPW, DIM), jnp.float32),
            pltpu.VMEM((BPW, DIM), jnp.float32),
            pltpu.VMEM((BPW, DIM), jnp.float32),
            pltpu.VMEM((BPW, DIM), jnp.float32),
            pltpu.VMEM((BPW, DIM), jnp.float32),
            pltpu.VMEM((BPW,), jnp.float32),
            pltpu.SemaphoreType.DMA,
        ],
    )
    def sc_kernel(h_hbm, r_hbm, t_hbm, ent_emb_hbm, ent_proj_hbm,
                  rel_emb_hbm, rel_proj_hbm, out_hbm,
                  hi_v, ti_v, ri_v, eh_v, hp_v, et_v, tp_v, er_v, rp_v,
                  out_v, sem):
        wid = lax.axis_index("s") * NC + lax.axis_index("c")
        base = wid * BPW

        for c in range(NCHUNK):
            src = pl.ds(base + c * CHUNK, CHUNK)
            pltpu.sync_copy(h_hbm.at[src], hi_v.at[c])
            pltpu.sync_copy(t_hbm.at[src], ti_v.at[c])
            pltpu.sync_copy(r_hbm.at[src], ri_v.at[c])

        copies = []
        for c in range(NCHUNK):
            dst = pl.ds(c * CHUNK, CHUNK)
            copies.append(pltpu.async_copy(
                ent_emb_hbm.at[hi_v.at[c]], eh_v.at[dst], sem))
            copies.append(pltpu.async_copy(
                ent_proj_hbm.at[hi_v.at[c]], hp_v.at[dst], sem))
            copies.append(pltpu.async_copy(
                ent_emb_hbm.at[ti_v.at[c]], et_v.at[dst], sem))
            copies.append(pltpu.async_copy(
                ent_proj_hbm.at[ti_v.at[c]], tp_v.at[dst], sem))
            copies.append(pltpu.async_copy(
                rel_emb_hbm.at[ri_v.at[c]], er_v.at[dst], sem))
            copies.append(pltpu.async_copy(
                rel_proj_hbm.at[ri_v.at[c]], rp_v.at[dst], sem))
        for cp in copies:
            cp.wait()

        lo = pl.ds(0, L)
        hi = pl.ds(L, L)
        lane = lax.iota(jnp.int32, L)

        def body(g, carry):
            sv = jnp.zeros((L,), jnp.float32)
            for j in range(L):
                e = g * L + j
                a0 = eh_v[e, lo]
                a1 = eh_v[e, hi]
                p0 = hp_v[e, lo]
                p1 = hp_v[e, hi]
                b0 = et_v[e, lo]
                b1 = et_v[e, hi]
                c0 = tp_v[e, lo]
                c1 = tp_v[e, hi]
                q0 = rp_v[e, lo]
                q1 = rp_v[e, hi]
                r0 = er_v[e, lo]
                r1 = er_v[e, hi]

                s_h = _vreduce_splat(a0 * p0 + a1 * p1, lane)
                s_t = _vreduce_splat(b0 * c0 + b1 * c1, lane)

                hb0 = q0 * s_h + a0
                hb1 = q1 * s_h + a1
                tb0 = q0 * s_t + b0
                tb1 = q1 * s_t + b1

                n_h = _vreduce_splat(hb0 * hb0 + hb1 * hb1, lane)
                n_t = _vreduce_splat(tb0 * tb0 + tb1 * tb1, lane)
                sc_h = _clamp_scale(n_h)
                sc_t = _clamp_scale(n_t)

                d0 = sc_h * hb0 + r0 - sc_t * tb0
                d1 = sc_h * hb1 + r1 - sc_t * tb1
                score = _vreduce_splat(d0 * d0 + d1 * d1, lane)
                sv = jnp.where(lane == j, score, sv)
            out_v[pl.ds(g * L, L)] = sv
            return carry

        lax.fori_loop(0, BPW // L, body, 0)
        pltpu.sync_copy(out_v, out_hbm.at[pl.ds(base, BPW)])

    return sc_kernel


_SC_KERNEL = _make_sc_kernel()


def kernel(h, r, t, ent_emb, ent_proj, rel_emb, rel_proj):
    h = h.astype(jnp.int32)
    r = r.astype(jnp.int32)
    t = t.astype(jnp.int32)
    return _SC_KERNEL(h, r, t, ent_emb, ent_proj, rel_emb, rel_proj)
